# baseline (speedup 1.0000x reference)
import jax
import jax.numpy as jnp
from jax import lax
from jax.experimental import pallas as pl
from jax.experimental.pallas import tpu as pltpu

_W = 64


def _windowed_sort_kernel(v_ref, o_ref):
    x = v_ref[0]
    L, C = x.shape
    ti = lax.broadcasted_iota(jnp.int32, (L, C), 0)
    ci = lax.broadcasted_iota(jnp.int32, (L, C), 1)
    r = (ti - ci) & (_W - 1)
    bit_zero = [(r & (1 << b)) == 0 for b in range(6)]

    k = 2
    while k <= _W:
        j = k // 2
        while j > 0:
            up = jnp.concatenate([x[j:], x[:j]], axis=0)
            dn = jnp.concatenate([x[-j:], x[:-j]], axis=0)
            bitj = bit_zero[j.bit_length() - 1]
            p = jnp.where(bitj, up, dn)
            if k == _W:
                take_min = bitj
            else:
                take_min = bit_zero[k.bit_length() - 1] == bitj
            x = jnp.where(take_min, jnp.minimum(x, p), jnp.maximum(x, p))
            j //= 2
        k *= 2
    o_ref[0] = x


def kernel(q, k, v):
    B, L, D = v.shape
    C = 512
    grid = (B, D // C)
    return pl.pallas_call(
        _windowed_sort_kernel,
        grid=grid,
        in_specs=[pl.BlockSpec((1, L, C), lambda b, c: (b, 0, c))],
        out_specs=pl.BlockSpec((1, L, C), lambda b, c: (b, 0, c)),
        out_shape=jax.ShapeDtypeStruct(v.shape, v.dtype),
        compiler_params=pltpu.CompilerParams(
            dimension_semantics=("parallel", "parallel"),
        ),
    )(v)


# speedup vs baseline: 1.2047x; 1.2047x over previous
import jax
import jax.numpy as jnp
from jax import lax
from jax.experimental import pallas as pl
from jax.experimental.pallas import tpu as pltpu

_W = 64


def _roll_up(z, sh):
    return jnp.concatenate([z[sh:], z[:sh]], axis=0)


def _shift_by_lane(z, amt_masks):
    for b, m in enumerate(amt_masks):
        sh = 1 << b
        z = jnp.where(m, _roll_up(z, sh), z)
    return z


def _sort64(w, row):
    k = 2
    while k <= _W:
        j = k // 2
        while j > 0:
            up = _roll_up(w, j)
            dn = _roll_up(w, _W - j)
            bitj = (row & j) == 0
            p = jnp.where(bitj, up, dn)
            if k == _W:
                take_min = bitj
            else:
                take_min = ((row & k) == 0) == bitj
            w = jnp.where(take_min, jnp.minimum(w, p), jnp.maximum(w, p))
            j //= 2
        k *= 2
    return w


def _windowed_sort_kernel(v_ref, o_ref, y_ref):
    x = v_ref[0]
    L, C = x.shape
    n_strips = L // _W
    lane = lax.broadcasted_iota(jnp.int32, (1, C), 1) & (_W - 1)
    row = lax.broadcasted_iota(jnp.int32, (_W, 1), 0)
    fwd_masks = [(lane & (1 << b)) != 0 for b in range(6)]
    amt = _W - lane
    inv_masks = [(amt & (1 << b)) != 0 for b in range(7)]

    def sort_strip(z):
        return _sort64(_shift_by_lane(z, fwd_masks)[:_W], row)

    def sort_body(s, _):
        z = v_ref[0, pl.ds(_W * s, 2 * _W), :]
        y_ref[pl.ds(_W * s, _W), :] = sort_strip(z)
        return 0

    lax.fori_loop(0, n_strips - 1, sort_body, 0)
    z_last = jnp.concatenate([x[L - _W:], x[:_W]], axis=0)
    y_ref[L - _W:, :] = sort_strip(z_last)

    def unshift_body(s, _):
        z = y_ref[pl.ds(_W * (s - 1), 2 * _W), :]
        o_ref[0, pl.ds(_W * s, _W), :] = _shift_by_lane(z, inv_masks)[:_W]
        return 0

    z0 = jnp.concatenate([y_ref[L - _W:, :], y_ref[: _W, :]], axis=0)
    o_ref[0, : _W, :] = _shift_by_lane(z0, inv_masks)[:_W]
    lax.fori_loop(1, n_strips, unshift_body, 0)


def kernel(q, k, v):
    B, L, D = v.shape
    C = 128
    grid = (B, D // C)
    return pl.pallas_call(
        _windowed_sort_kernel,
        grid=grid,
        in_specs=[pl.BlockSpec((1, L, C), lambda b, c: (b, 0, c))],
        out_specs=pl.BlockSpec((1, L, C), lambda b, c: (b, 0, c)),
        out_shape=jax.ShapeDtypeStruct(v.shape, v.dtype),
        scratch_shapes=[pltpu.VMEM((L, C), jnp.float32)],
        compiler_params=pltpu.CompilerParams(
            dimension_semantics=("parallel", "parallel"),
        ),
    )(v)


# speedup vs baseline: 1.3507x; 1.1212x over previous
import jax
import jax.numpy as jnp
from jax import lax
from jax.experimental import pallas as pl
from jax.experimental.pallas import tpu as pltpu

_W = 64


def _roll_up(z, sh):
    return jnp.concatenate([z[sh:], z[:sh]], axis=0)


def _shift_by_lane(z, amt_masks):
    for b, m in enumerate(amt_masks):
        sh = 1 << b
        z = jnp.where(m, _roll_up(z, sh), z)
    return z


def _sort64(w, row):
    k = 2
    while k <= _W:
        j = k // 2
        while j > 0:
            up = _roll_up(w, j)
            dn = _roll_up(w, _W - j)
            bitj = (row & j) == 0
            p = jnp.where(bitj, up, dn)
            if k == _W:
                take_min = bitj
            else:
                take_min = ((row & k) == 0) == bitj
            w = jnp.where(take_min, jnp.minimum(w, p), jnp.maximum(w, p))
            j //= 2
        k *= 2
    return w


def _windowed_sort_kernel(v_ref, o_ref, y_ref):
    x = v_ref[0]
    L, C = x.shape
    n_strips = L // _W
    lane = lax.broadcasted_iota(jnp.int32, (1, C), 1) & (_W - 1)
    row = lax.broadcasted_iota(jnp.int32, (_W, 1), 0)
    fwd_masks = [(lane & (1 << b)) != 0 for b in range(6)]
    amt = _W - lane
    inv_masks = [(amt & (1 << b)) != 0 for b in range(7)]

    def sort_strip(z):
        return _sort64(_shift_by_lane(z, fwd_masks)[:_W], row)

    def sort_body(s, _):
        z = v_ref[0, pl.ds(_W * s, 2 * _W), :]
        y_ref[pl.ds(_W * s, _W), :] = sort_strip(z)
        return 0

    lax.fori_loop(0, n_strips - 1, sort_body, 0)
    z_last = jnp.concatenate([x[L - _W:], x[:_W]], axis=0)
    y_ref[L - _W:, :] = sort_strip(z_last)

    def unshift_body(s, _):
        z = y_ref[pl.ds(_W * (s - 1), 2 * _W), :]
        o_ref[0, pl.ds(_W * s, _W), :] = _shift_by_lane(z, inv_masks)[:_W]
        return 0

    z0 = jnp.concatenate([y_ref[L - _W:, :], y_ref[: _W, :]], axis=0)
    o_ref[0, : _W, :] = _shift_by_lane(z0, inv_masks)[:_W]
    lax.fori_loop(1, n_strips, unshift_body, 0)


def kernel(q, k, v):
    B, L, D = v.shape
    C = 256
    grid = (B, D // C)
    return pl.pallas_call(
        _windowed_sort_kernel,
        grid=grid,
        in_specs=[pl.BlockSpec((1, L, C), lambda b, c: (b, 0, c))],
        out_specs=pl.BlockSpec((1, L, C), lambda b, c: (b, 0, c)),
        out_shape=jax.ShapeDtypeStruct(v.shape, v.dtype),
        scratch_shapes=[pltpu.VMEM((L, C), jnp.float32)],
        compiler_params=pltpu.CompilerParams(
            dimension_semantics=("parallel", "parallel"),
        ),
    )(v)
